# baseline (speedup 1.0000x reference)
import functools
import jax
import jax.numpy as jnp
from jax import lax
from jax.experimental import pallas as pl
from jax.experimental.pallas import tpu as pltpu
from jax.experimental.pallas import tpu_sc as plsc

D_MODEL = 64
LANES = 128
NUM_CORES = 2
NUM_SUBCORES = 16
NUM_WORKERS = NUM_CORES * NUM_SUBCORES
SEQ = 200
NBATCH = SEQ // 2


def _body(x4d_hbm, te_hbm, out_hbm, idx_v, rows_v, tile_v, gsem, wsem, lsem):
    tb = lax.axis_index("s") * NUM_CORES + lax.axis_index("c")

    base_iota = lax.iota(jnp.int32, 16)
    perm = [lax.bitwise_and(base_iota + c, 15) for c in range(16)]
    perm_hi = [lax.shift_right_logical(p, 3) for p in perm]
    perm_lo = [lax.bitwise_and(p, 7) for p in perm]

    pltpu.async_copy(x4d_hbm.at[:, tb], idx_v, lsem).wait()

    def idx_slice(q, j):
        return idx_v.at[lax.shift_right_logical(q, 2),
                        2 * lax.bitwise_and(q, 3) + j]

    def start_gather(q, b):
        for j in range(2):
            pltpu.async_copy(te_hbm.at[idx_slice(q, j)], rows_v.at[b, j],
                             gsem.at[b])

    def wait_gather(q, b):
        for j in range(2):
            pltpu.make_async_copy(te_hbm.at[idx_slice(q, j)],
                                  rows_v.at[b, j], gsem.at[b]).wait()

    def transpose(b):
        for j in range(2):
            A = rows_v.at[b, j]
            B = tile_v.at[b, j]

            @plsc.parallel_loop(0, 32, unroll=2)
            def _(blk):
                row0 = lax.shift_right_logical(blk, 2) * 16
                col0 = lax.bitwise_and(blk, 3) * 16
                col0_hi = lax.bitwise_and(blk, 3) * 2
                rvec = base_iota + row0
                for c in range(16):
                    cvec = perm[c] + col0
                    v = plsc.load_gather(A, [rvec, cvec])
                    plsc.store_scatter(B, [perm_hi[c] + col0_hi, perm_lo[c],
                                           rvec], v)

    def out_slice(q):
        return out_hbm.at[pl.ds(2 * q, 2), :, tb]

    def start_wo(q, b):
        pltpu.async_copy(tile_v.at[b], out_slice(q), wsem.at[b])

    def wait_wo(q, b):
        pltpu.make_async_copy(tile_v.at[b], out_slice(q), wsem.at[b]).wait()

    start_gather(0, 0)
    start_gather(1, 1)

    def batch(q, b, first=False, last=False):
        wait_gather(q, b)
        if not first:
            wait_wo(q - 2, b)
        transpose(b)
        start_wo(q, b)
        if not last:
            start_gather(q + 2, b)

    batch(0, 0, first=True)
    batch(1, 1, first=True)

    def outer(qo, carry):
        q = qo * 2
        batch(q, 0)
        batch(q + 1, 1)
        return carry

    lax.fori_loop(1, NBATCH // 2 - 1, outer, 0, unroll=False)

    batch(NBATCH - 2, 0, last=True)
    batch(NBATCH - 1, 1, last=True)
    wait_wo(NBATCH - 2, 0)
    wait_wo(NBATCH - 1, 1)


def kernel(x, te):
    batch, seq = x.shape
    assert batch % LANES == 0 and seq == SEQ and D_MODEL == te.shape[1]
    n_btiles = batch // LANES

    x4d = (x.astype(jnp.int32).T.reshape(seq // 8, 8, n_btiles, LANES)
           .transpose((0, 2, 1, 3)))

    mesh = plsc.VectorSubcoreMesh(core_axis_name="c", subcore_axis_name="s")
    run = pl.kernel(
        _body,
        out_type=jax.ShapeDtypeStruct((seq, 8, n_btiles, 8, LANES),
                                      jnp.float32),
        mesh=mesh,
        scratch_types=[
            pltpu.VMEM((seq // 8, 8, LANES), jnp.int32),
            pltpu.VMEM((2, 2, LANES, D_MODEL), jnp.float32),
            pltpu.VMEM((2, 2, 8, 8, LANES), jnp.float32),
            pltpu.SemaphoreType.DMA((2,)),
            pltpu.SemaphoreType.DMA((2,)),
            pltpu.SemaphoreType.DMA,
        ],
        compiler_params=pltpu.CompilerParams(use_tc_tiling_on_sc=False,
                                             needs_layout_passes=False),
    )
    out5 = run(x4d, te)
    return out5.transpose((2, 4, 0, 1, 3)).reshape(batch, seq, D_MODEL)
